# Initial kernel scaffold; baseline (speedup 1.0000x reference)
import jax
import jax.numpy as jnp
from jax.experimental import pallas as pl


def kernel(x, edge_index, edge_attr, u, W, b):
    raise NotImplementedError("write your pallas kernel here")


# speedup vs baseline: 3.0745x; 3.0745x over previous
import functools

import jax
import jax.numpy as jnp
from jax import lax
from jax.experimental import pallas as pl
from jax.experimental.pallas import tpu as pltpu
from jax.experimental.pallas import tpu_sc as plsc

N = 10000
E = 320000
D = 128
DE = 16
DOUT = 16

NC = 2
NS = 16
NW = NC * NS
CH = 128
NCH = 80
E_PAD = NW * NCH * CH


def _proj_body(x_ref, wr_ref, ws_ref, xr_ref, xs_ref):
    xb = x_ref[...]
    xr_ref[...] = jnp.dot(xb, wr_ref[...], preferred_element_type=jnp.float32)
    xs_ref[...] = jnp.dot(xb, ws_ref[...], preferred_element_type=jnp.float32)


def _project(x, wr, ws):
    return pl.pallas_call(
        _proj_body,
        grid=(10,),
        in_specs=[
            pl.BlockSpec((N // 10, D), lambda i: (i, 0)),
            pl.BlockSpec((D, DOUT), lambda i: (0, 0)),
            pl.BlockSpec((D, DOUT), lambda i: (0, 0)),
        ],
        out_specs=[
            pl.BlockSpec((N // 10, DOUT), lambda i: (i, 0)),
            pl.BlockSpec((N // 10, DOUT), lambda i: (i, 0)),
        ],
        out_shape=[
            jax.ShapeDtypeStruct((N, DOUT), jnp.float32),
            jax.ShapeDtypeStruct((N, DOUT), jnp.float32),
        ],
    )(x, wr, ws)


def _gather_sum(dst_idx, src_idx, xr, xs):
    mesh = plsc.VectorSubcoreMesh(core_axis_name="c", subcore_axis_name="s")

    @functools.partial(
        pl.kernel,
        out_type=jax.ShapeDtypeStruct((E_PAD, DOUT), jnp.float32),
        mesh=mesh,
        scratch_types=[
            pltpu.VMEM((NCH, CH), jnp.int32),
            pltpu.VMEM((NCH, CH), jnp.int32),
            pltpu.VMEM((CH, DOUT), jnp.float32),
            pltpu.SemaphoreType.DMA,
        ],
        compiler_params=pltpu.CompilerParams(use_tc_tiling_on_sc=False),
    )
    def sc_kernel(dst_hbm, src_hbm, xr_hbm, xs_hbm, g_hbm, idxd, idxs, acc, sem):
        wid = lax.axis_index("s") * NC + lax.axis_index("c")
        pltpu.sync_copy(dst_hbm.at[wid], idxd)
        pltpu.sync_copy(src_hbm.at[wid], idxs)
        base = wid * (NCH * CH)

        def chunk(j, carry):
            pltpu.async_copy(xr_hbm.at[idxd.at[j]], acc, sem).wait()
            pltpu.async_copy(xs_hbm.at[idxs.at[j]], acc, sem, add=True).wait()
            pltpu.sync_copy(acc, g_hbm.at[pl.ds(base + j * CH, CH)])
            return carry

        lax.fori_loop(0, NCH, chunk, 0)

    return sc_kernel(dst_idx, src_idx, xr, xs)


def _final_body(ea_ref, g_ref, we_ref, wu_ref, u_ref, b_ref, out_ref):
    const = u_ref[0, 0] * wu_ref[...] + b_ref[...]
    out_ref[...] = (
        jnp.dot(ea_ref[...], we_ref[...], preferred_element_type=jnp.float32)
        + g_ref[...]
        + const
    )


def _finalize(edge_attr, g_pad, we, wu, u, b):
    blk = 4000
    return pl.pallas_call(
        _final_body,
        grid=(E // blk,),
        in_specs=[
            pl.BlockSpec((blk, DE), lambda i: (i, 0)),
            pl.BlockSpec((blk, DOUT), lambda i: (i, 0)),
            pl.BlockSpec((DE, DOUT), lambda i: (0, 0)),
            pl.BlockSpec((1, DOUT), lambda i: (0, 0)),
            pl.BlockSpec((1, 1), lambda i: (0, 0), memory_space=pltpu.SMEM),
            pl.BlockSpec((1, DOUT), lambda i: (0, 0)),
        ],
        out_specs=pl.BlockSpec((blk, DOUT), lambda i: (i, 0)),
        out_shape=jax.ShapeDtypeStruct((E, DOUT), jnp.float32),
    )(edge_attr, g_pad, we, wu, u, b)


def kernel(x, edge_index, edge_attr, u, W, b):
    wr = W[DE:DE + D]
    ws = W[DE + D:DE + 2 * D]
    we = W[:DE]
    wu = W[DE + 2 * D:]

    xr, xs = _project(x, wr, ws)

    idx = jnp.zeros((2, E_PAD), jnp.int32).at[:, :E].set(edge_index)
    dst_idx = idx[1].reshape(NW, NCH, CH)
    src_idx = idx[0].reshape(NW, NCH, CH)

    g_pad = _gather_sum(dst_idx, src_idx, xr, xs)

    return _finalize(
        edge_attr,
        g_pad,
        we,
        wu,
        u.reshape(1, 1),
        b.reshape(1, DOUT),
    )


# speedup vs baseline: 3.3157x; 1.0785x over previous
import functools

import jax
import jax.numpy as jnp
from jax import lax
from jax.experimental import pallas as pl
from jax.experimental.pallas import tpu as pltpu
from jax.experimental.pallas import tpu_sc as plsc

N = 10000
E = 320000
D = 128
DE = 16
DOUT = 16

NC = 2
NS = 16
NW = NC * NS
CH = 128
NCH = 80
E_PAD = NW * NCH * CH


def _proj_body(x_ref, wr_ref, ws_ref, xr_ref, xs_ref):
    xb = x_ref[...]
    xr_ref[...] = jnp.dot(xb, wr_ref[...], preferred_element_type=jnp.float32)
    xs_ref[...] = jnp.dot(xb, ws_ref[...], preferred_element_type=jnp.float32)


def _project(x, wr, ws):
    return pl.pallas_call(
        _proj_body,
        grid=(10,),
        in_specs=[
            pl.BlockSpec((N // 10, D), lambda i: (i, 0)),
            pl.BlockSpec((D, DOUT), lambda i: (0, 0)),
            pl.BlockSpec((D, DOUT), lambda i: (0, 0)),
        ],
        out_specs=[
            pl.BlockSpec((N // 10, DOUT), lambda i: (i, 0)),
            pl.BlockSpec((N // 10, DOUT), lambda i: (i, 0)),
        ],
        out_shape=[
            jax.ShapeDtypeStruct((N, DOUT), jnp.float32),
            jax.ShapeDtypeStruct((N, DOUT), jnp.float32),
        ],
    )(x, wr, ws)


NB = 20
NWAVE = NCH // NB


def _gather_sum(dst_idx, src_idx, xr, xs):
    mesh = plsc.VectorSubcoreMesh(core_axis_name="c", subcore_axis_name="s")

    @functools.partial(
        pl.kernel,
        out_type=jax.ShapeDtypeStruct((E_PAD, DOUT), jnp.float32),
        mesh=mesh,
        scratch_types=[
            pltpu.VMEM((NCH, CH), jnp.int32),
            pltpu.VMEM((NCH, CH), jnp.int32),
            pltpu.VMEM((NB * CH, DOUT), jnp.float32),
            pltpu.SemaphoreType.DMA,
        ],
        compiler_params=pltpu.CompilerParams(use_tc_tiling_on_sc=False),
    )
    def sc_kernel(dst_hbm, src_hbm, xr_hbm, xs_hbm, g_hbm, idxd, idxs, acc, sem):
        wid = lax.axis_index("s") * NC + lax.axis_index("c")
        pltpu.sync_copy(dst_hbm.at[wid], idxd)
        pltpu.sync_copy(src_hbm.at[wid], idxs)
        base = wid * (NCH * CH)

        def wave(w, carry):
            off = base + w * (NB * CH)
            drain = pltpu.make_async_copy(
                g_hbm.at[pl.ds(off, NB * CH)], acc, sem
            )

            def fire_r(b, c):
                pltpu.async_copy(
                    xr_hbm.at[idxd.at[w * NB + b]],
                    acc.at[pl.ds(b * CH, CH)],
                    sem,
                )
                return c

            lax.fori_loop(0, NB, fire_r, 0)
            drain.wait()

            def fire_s(b, c):
                pltpu.async_copy(
                    xs_hbm.at[idxs.at[w * NB + b]],
                    acc.at[pl.ds(b * CH, CH)],
                    sem,
                    add=True,
                )
                return c

            lax.fori_loop(0, NB, fire_s, 0)
            drain.wait()
            pltpu.sync_copy(acc, g_hbm.at[pl.ds(off, NB * CH)])
            return carry

        lax.fori_loop(0, NWAVE, wave, 0)

    return sc_kernel(dst_idx, src_idx, xr, xs)


def _final_body(ea_ref, g_ref, we_ref, wu_ref, u_ref, b_ref, out_ref):
    const = u_ref[0, 0] * wu_ref[...] + b_ref[...]
    out_ref[...] = (
        jnp.dot(ea_ref[...], we_ref[...], preferred_element_type=jnp.float32)
        + g_ref[...]
        + const
    )


def _finalize(edge_attr, g_pad, we, wu, u, b):
    blk = 4000
    return pl.pallas_call(
        _final_body,
        grid=(E // blk,),
        in_specs=[
            pl.BlockSpec((blk, DE), lambda i: (i, 0)),
            pl.BlockSpec((blk, DOUT), lambda i: (i, 0)),
            pl.BlockSpec((DE, DOUT), lambda i: (0, 0)),
            pl.BlockSpec((1, DOUT), lambda i: (0, 0)),
            pl.BlockSpec((1, 1), lambda i: (0, 0), memory_space=pltpu.SMEM),
            pl.BlockSpec((1, DOUT), lambda i: (0, 0)),
        ],
        out_specs=pl.BlockSpec((blk, DOUT), lambda i: (i, 0)),
        out_shape=jax.ShapeDtypeStruct((E, DOUT), jnp.float32),
    )(edge_attr, g_pad, we, wu, u, b)


def kernel(x, edge_index, edge_attr, u, W, b):
    wr = W[DE:DE + D]
    ws = W[DE + D:DE + 2 * D]
    we = W[:DE]
    wu = W[DE + 2 * D:]

    xr, xs = _project(x, wr, ws)

    idx = jnp.zeros((2, E_PAD), jnp.int32).at[:, :E].set(edge_index)
    dst_idx = idx[1].reshape(NW, NCH, CH)
    src_idx = idx[0].reshape(NW, NCH, CH)

    g_pad = _gather_sum(dst_idx, src_idx, xr, xs)

    return _finalize(
        edge_attr,
        g_pad,
        we,
        wu,
        u.reshape(1, 1),
        b.reshape(1, DOUT),
    )


# speedup vs baseline: 3.3282x; 1.0038x over previous
import functools

import jax
import jax.numpy as jnp
from jax import lax
from jax.experimental import pallas as pl
from jax.experimental.pallas import tpu as pltpu
from jax.experimental.pallas import tpu_sc as plsc

N = 10000
E = 320000
D = 128
DE = 16
DOUT = 16

NC = 2
NS = 16
NW = NC * NS
CH = 128
NCH = 80
E_PAD = NW * NCH * CH


def _proj_body(x_ref, wr_ref, ws_ref, xr_ref, xs_ref):
    xb = x_ref[...]
    xr_ref[...] = jnp.dot(xb, wr_ref[...], preferred_element_type=jnp.float32)
    xs_ref[...] = jnp.dot(xb, ws_ref[...], preferred_element_type=jnp.float32)


def _project(x, wr, ws):
    return pl.pallas_call(
        _proj_body,
        grid=(10,),
        in_specs=[
            pl.BlockSpec((N // 10, D), lambda i: (i, 0)),
            pl.BlockSpec((D, DOUT), lambda i: (0, 0)),
            pl.BlockSpec((D, DOUT), lambda i: (0, 0)),
        ],
        out_specs=[
            pl.BlockSpec((N // 10, DOUT), lambda i: (i, 0)),
            pl.BlockSpec((N // 10, DOUT), lambda i: (i, 0)),
        ],
        out_shape=[
            jax.ShapeDtypeStruct((N, DOUT), jnp.float32),
            jax.ShapeDtypeStruct((N, DOUT), jnp.float32),
        ],
    )(x, wr, ws)


NB = 20
NWAVE = NCH // NB


def _gather_sum(dst_idx, src_idx, xr, xs):
    mesh = plsc.VectorSubcoreMesh(core_axis_name="c", subcore_axis_name="s")

    @functools.partial(
        pl.kernel,
        out_type=jax.ShapeDtypeStruct((E_PAD, DOUT), jnp.float32),
        mesh=mesh,
        scratch_types=[
            pltpu.VMEM((NCH * CH,), jnp.int32),
            pltpu.VMEM((NCH * CH,), jnp.int32),
            pltpu.VMEM((NB * CH, DOUT), jnp.float32),
            pltpu.SemaphoreType.DMA,
        ],
        compiler_params=pltpu.CompilerParams(use_tc_tiling_on_sc=False),
    )
    def sc_kernel(dst_hbm, src_hbm, xr_hbm, xs_hbm, g_hbm, idxd, idxs, acc, sem):
        wid = lax.axis_index("s") * NC + lax.axis_index("c")
        pltpu.sync_copy(dst_hbm.at[wid], idxd)
        pltpu.sync_copy(src_hbm.at[wid], idxs)
        base = wid * (NCH * CH)
        wch = NB * CH

        def wave(w, carry):
            off = base + w * wch
            pltpu.async_copy(
                xr_hbm.at[idxd.at[pl.ds(w * wch, wch)]], acc, sem
            ).wait()
            pltpu.async_copy(
                xs_hbm.at[idxs.at[pl.ds(w * wch, wch)]], acc, sem, add=True
            ).wait()
            pltpu.sync_copy(acc, g_hbm.at[pl.ds(off, wch)])
            return carry

        lax.fori_loop(0, NWAVE, wave, 0)

    return sc_kernel(dst_idx, src_idx, xr, xs)


def _final_body(ea_ref, g_ref, we_ref, wu_ref, u_ref, b_ref, out_ref):
    const = u_ref[0, 0] * wu_ref[...] + b_ref[...]
    out_ref[...] = (
        jnp.dot(ea_ref[...], we_ref[...], preferred_element_type=jnp.float32)
        + g_ref[...]
        + const
    )


def _finalize(edge_attr, g_pad, we, wu, u, b):
    blk = 4000
    return pl.pallas_call(
        _final_body,
        grid=(E // blk,),
        in_specs=[
            pl.BlockSpec((blk, DE), lambda i: (i, 0)),
            pl.BlockSpec((blk, DOUT), lambda i: (i, 0)),
            pl.BlockSpec((DE, DOUT), lambda i: (0, 0)),
            pl.BlockSpec((1, DOUT), lambda i: (0, 0)),
            pl.BlockSpec((1, 1), lambda i: (0, 0), memory_space=pltpu.SMEM),
            pl.BlockSpec((1, DOUT), lambda i: (0, 0)),
        ],
        out_specs=pl.BlockSpec((blk, DOUT), lambda i: (i, 0)),
        out_shape=jax.ShapeDtypeStruct((E, DOUT), jnp.float32),
    )(edge_attr, g_pad, we, wu, u, b)


def kernel(x, edge_index, edge_attr, u, W, b):
    wr = W[DE:DE + D]
    ws = W[DE + D:DE + 2 * D]
    we = W[:DE]
    wu = W[DE + 2 * D:]

    xr, xs = _project(x, wr, ws)

    idx = jnp.zeros((2, E_PAD), jnp.int32).at[:, :E].set(edge_index)
    dst_idx = idx[1].reshape(NW, NCH * CH)
    src_idx = idx[0].reshape(NW, NCH * CH)

    g_pad = _gather_sum(dst_idx, src_idx, xr, xs)

    return _finalize(
        edge_attr,
        g_pad,
        we,
        wu,
        u.reshape(1, 1),
        b.reshape(1, DOUT),
    )


# speedup vs baseline: 4.5371x; 1.3632x over previous
import functools

import jax
import jax.numpy as jnp
from jax import lax
from jax.experimental import pallas as pl
from jax.experimental.pallas import tpu as pltpu
from jax.experimental.pallas import tpu_sc as plsc

N = 10000
E = 320000
D = 128
DE = 16
DOUT = 16

NC = 2
NS = 16
NW = NC * NS
CH = 128
NCH = 80
E_PAD = NW * NCH * CH


def _proj_body(x_ref, wr_ref, ws_ref, xr_ref, xs_ref):
    xb = x_ref[...]
    xr_ref[...] = jnp.dot(xb, wr_ref[...], preferred_element_type=jnp.float32)
    xs_ref[...] = jnp.dot(xb, ws_ref[...], preferred_element_type=jnp.float32)


def _project(x, wr, ws):
    return pl.pallas_call(
        _proj_body,
        grid=(10,),
        in_specs=[
            pl.BlockSpec((N // 10, D), lambda i: (i, 0)),
            pl.BlockSpec((D, DOUT), lambda i: (0, 0)),
            pl.BlockSpec((D, DOUT), lambda i: (0, 0)),
        ],
        out_specs=[
            pl.BlockSpec((N // 10, DOUT), lambda i: (i, 0)),
            pl.BlockSpec((N // 10, DOUT), lambda i: (i, 0)),
        ],
        out_shape=[
            jax.ShapeDtypeStruct((N, DOUT), jnp.float32),
            jax.ShapeDtypeStruct((N, DOUT), jnp.float32),
        ],
    )(x, wr, ws)


NB = 20
NWAVE = NCH // NB


def _gather_sum(dst_idx, src_idx, xr, xs):
    mesh = plsc.VectorSubcoreMesh(core_axis_name="c", subcore_axis_name="s")

    @functools.partial(
        pl.kernel,
        out_type=jax.ShapeDtypeStruct((E_PAD, DOUT), jnp.float32),
        mesh=mesh,
        scratch_types=[
            pltpu.VMEM((NCH * CH,), jnp.int32),
            pltpu.VMEM((NCH * CH,), jnp.int32),
            pltpu.VMEM((NB * CH, DOUT), jnp.float32),
            pltpu.SemaphoreType.DMA,
        ],
        compiler_params=pltpu.CompilerParams(use_tc_tiling_on_sc=False),
    )
    def sc_kernel(dst_hbm, src_hbm, xr_hbm, xs_hbm, g_hbm, idxd, idxs, acc, sem):
        wid = lax.axis_index("s") * NC + lax.axis_index("c")
        pltpu.sync_copy(dst_hbm.at[wid], idxd)
        pltpu.sync_copy(src_hbm.at[wid], idxs)
        base = wid * (NCH * CH)
        wch = NB * CH

        def wave(w, carry):
            off = base + w * wch
            pltpu.async_copy(
                xr_hbm.at[idxd.at[pl.ds(w * wch, wch)]], acc, sem
            ).wait()
            pltpu.async_copy(
                xs_hbm.at[idxs.at[pl.ds(w * wch, wch)]], acc, sem, add=True
            ).wait()
            pltpu.sync_copy(acc, g_hbm.at[pl.ds(off, wch)])
            return carry

        lax.fori_loop(0, NWAVE, wave, 0)

    return sc_kernel(dst_idx, src_idx, xr, xs)


def _final_body(ea_ref, g_ref, we_ref, wu_ref, u_ref, b_ref, out_ref):
    const = u_ref[0, 0] * wu_ref[...] + b_ref[...]
    out_ref[...] = (
        jnp.dot(ea_ref[...], we_ref[...], preferred_element_type=jnp.float32)
        + g_ref[...]
        + const
    )


def _finalize(ea2, g2, we_big, wu_big, u, b_big):
    blk = 6400
    return pl.pallas_call(
        _final_body,
        grid=(E // blk,),
        in_specs=[
            pl.BlockSpec((blk // 8, 8 * DE), lambda i: (i, 0)),
            pl.BlockSpec((blk // 8, 8 * DOUT), lambda i: (i, 0)),
            pl.BlockSpec((8 * DE, 8 * DOUT), lambda i: (0, 0)),
            pl.BlockSpec((1, 8 * DOUT), lambda i: (0, 0)),
            pl.BlockSpec((1, 1), lambda i: (0, 0), memory_space=pltpu.SMEM),
            pl.BlockSpec((1, 8 * DOUT), lambda i: (0, 0)),
        ],
        out_specs=pl.BlockSpec((blk // 8, 8 * DOUT), lambda i: (i, 0)),
        out_shape=jax.ShapeDtypeStruct((E // 8, 8 * DOUT), jnp.float32),
    )(ea2, g2, we_big, wu_big, u, b_big)


def kernel(x, edge_index, edge_attr, u, W, b):
    wr = W[DE:DE + D]
    ws = W[DE + D:DE + 2 * D]
    we = W[:DE]
    wu = W[DE + 2 * D:]

    xr, xs = _project(x, wr, ws)

    idx = jnp.zeros((2, E_PAD), jnp.int32).at[:, :E].set(edge_index)
    dst_idx = idx[1].reshape(NW, NCH * CH)
    src_idx = idx[0].reshape(NW, NCH * CH)

    g_pad = _gather_sum(dst_idx, src_idx, xr, xs)

    eye8 = jnp.eye(8, dtype=jnp.float32)
    out2 = _finalize(
        edge_attr.reshape(E // 8, 8 * DE),
        g_pad.reshape(E_PAD // 8, 8 * DOUT),
        jnp.kron(eye8, we),
        jnp.tile(wu, (1, 8)),
        u.reshape(1, 1),
        jnp.tile(b.reshape(1, DOUT), (1, 8)),
    )
    return out2.reshape(E, DOUT)


# speedup vs baseline: 5.4684x; 1.2053x over previous
import functools

import jax
import jax.numpy as jnp
from jax import lax
from jax.experimental import pallas as pl
from jax.experimental.pallas import tpu as pltpu
from jax.experimental.pallas import tpu_sc as plsc

N = 10000
E = 320000
D = 128
DE = 16
DOUT = 16

NC = 2
NS = 16
NW = NC * NS
CH = 128
NCH = 80
E_PAD = NW * NCH * CH


def _proj_body(x_ref, wr_ref, ws_ref, xr_ref, xs_ref):
    xb = x_ref[...]
    xr_ref[...] = jnp.dot(xb, wr_ref[...], preferred_element_type=jnp.float32)
    xs_ref[...] = jnp.dot(xb, ws_ref[...], preferred_element_type=jnp.float32)


def _project(x, wr, ws):
    return pl.pallas_call(
        _proj_body,
        grid=(10,),
        in_specs=[
            pl.BlockSpec((N // 10, D), lambda i: (i, 0)),
            pl.BlockSpec((D, DOUT), lambda i: (0, 0)),
            pl.BlockSpec((D, DOUT), lambda i: (0, 0)),
        ],
        out_specs=[
            pl.BlockSpec((N // 10, DOUT), lambda i: (i, 0)),
            pl.BlockSpec((N // 10, DOUT), lambda i: (i, 0)),
        ],
        out_shape=[
            jax.ShapeDtypeStruct((N, DOUT), jnp.float32),
            jax.ShapeDtypeStruct((N, DOUT), jnp.float32),
        ],
    )(x, wr, ws)


NB = 20
NWAVE = NCH // NB


def _gather_sum(dst_idx, src_idx, xr, xs):
    mesh = plsc.VectorSubcoreMesh(core_axis_name="c", subcore_axis_name="s")

    @functools.partial(
        pl.kernel,
        out_type=jax.ShapeDtypeStruct((E_PAD, DOUT), jnp.float32),
        mesh=mesh,
        scratch_types=[
            pltpu.VMEM((NCH * CH,), jnp.int32),
            pltpu.VMEM((NCH * CH,), jnp.int32),
            pltpu.VMEM((NB * CH, DOUT), jnp.float32),
            pltpu.VMEM_SHARED((N, DOUT), jnp.float32),
            pltpu.VMEM_SHARED((N, DOUT), jnp.float32),
            pltpu.SemaphoreType.DMA,
        ],
        compiler_params=pltpu.CompilerParams(use_tc_tiling_on_sc=False),
    )
    def sc_kernel(dst_hbm, src_hbm, xr_hbm, xs_hbm, g_hbm,
                  idxd, idxs, acc, xr_sh, xs_sh, sem):
        sid = lax.axis_index("s")
        wid = sid * NC + lax.axis_index("c")
        rows = N // NS
        pltpu.sync_copy(
            xr_hbm.at[pl.ds(sid * rows, rows)],
            xr_sh.at[pl.ds(sid * rows, rows)],
        )
        pltpu.sync_copy(
            xs_hbm.at[pl.ds(sid * rows, rows)],
            xs_sh.at[pl.ds(sid * rows, rows)],
        )
        pltpu.sync_copy(dst_hbm.at[wid], idxd)
        pltpu.sync_copy(src_hbm.at[wid], idxs)
        plsc.subcore_barrier()
        base = wid * (NCH * CH)
        wch = NB * CH

        def wave(w, carry):
            off = base + w * wch
            pltpu.async_copy(
                xr_sh.at[idxd.at[pl.ds(w * wch, wch)]], acc, sem
            ).wait()
            pltpu.async_copy(
                xs_sh.at[idxs.at[pl.ds(w * wch, wch)]], acc, sem, add=True
            ).wait()
            pltpu.sync_copy(acc, g_hbm.at[pl.ds(off, wch)])
            return carry

        lax.fori_loop(0, NWAVE, wave, 0)

    return sc_kernel(dst_idx, src_idx, xr, xs)


def _final_body(ea_ref, g_ref, we_ref, wu_ref, u_ref, b_ref, out_ref):
    const = u_ref[0, 0] * wu_ref[...] + b_ref[...]
    out_ref[...] = (
        jnp.dot(ea_ref[...], we_ref[...], preferred_element_type=jnp.float32)
        + g_ref[...]
        + const
    )


def _finalize(ea2, g2, we_big, wu_big, u, b_big):
    blk = 6400
    return pl.pallas_call(
        _final_body,
        grid=(E // blk,),
        in_specs=[
            pl.BlockSpec((blk // 8, 8 * DE), lambda i: (i, 0)),
            pl.BlockSpec((blk // 8, 8 * DOUT), lambda i: (i, 0)),
            pl.BlockSpec((8 * DE, 8 * DOUT), lambda i: (0, 0)),
            pl.BlockSpec((1, 8 * DOUT), lambda i: (0, 0)),
            pl.BlockSpec((1, 1), lambda i: (0, 0), memory_space=pltpu.SMEM),
            pl.BlockSpec((1, 8 * DOUT), lambda i: (0, 0)),
        ],
        out_specs=pl.BlockSpec((blk // 8, 8 * DOUT), lambda i: (i, 0)),
        out_shape=jax.ShapeDtypeStruct((E // 8, 8 * DOUT), jnp.float32),
    )(ea2, g2, we_big, wu_big, u, b_big)


def kernel(x, edge_index, edge_attr, u, W, b):
    wr = W[DE:DE + D]
    ws = W[DE + D:DE + 2 * D]
    we = W[:DE]
    wu = W[DE + 2 * D:]

    xr, xs = _project(x, wr, ws)

    idx = jnp.zeros((2, E_PAD), jnp.int32).at[:, :E].set(edge_index)
    dst_idx = idx[1].reshape(NW, NCH * CH)
    src_idx = idx[0].reshape(NW, NCH * CH)

    g_pad = _gather_sum(dst_idx, src_idx, xr, xs)

    eye8 = jnp.eye(8, dtype=jnp.float32)
    out2 = _finalize(
        edge_attr.reshape(E // 8, 8 * DE),
        g_pad.reshape(E_PAD // 8, 8 * DOUT),
        jnp.kron(eye8, we),
        jnp.tile(wu, (1, 8)),
        u.reshape(1, 1),
        jnp.tile(b.reshape(1, DOUT), (1, 8)),
    )
    return out2.reshape(E, DOUT)


# speedup vs baseline: 5.7816x; 1.0573x over previous
import functools

import jax
import jax.numpy as jnp
from jax import lax
from jax.experimental import pallas as pl
from jax.experimental.pallas import tpu as pltpu
from jax.experimental.pallas import tpu_sc as plsc

N = 10000
E = 320000
D = 128
DE = 16
DOUT = 16

NC = 2
NS = 16
NW = NC * NS
CH = 128
NCH = 80
E_PAD = NW * NCH * CH


def _proj_body(x_ref, wr_ref, ws_ref, xr_ref, xs_ref):
    xb = x_ref[...]
    xr_ref[...] = jnp.dot(xb, wr_ref[...], preferred_element_type=jnp.float32)
    xs_ref[...] = jnp.dot(xb, ws_ref[...], preferred_element_type=jnp.float32)


def _project(x, wr, ws):
    return pl.pallas_call(
        _proj_body,
        grid=(10,),
        in_specs=[
            pl.BlockSpec((N // 10, D), lambda i: (i, 0)),
            pl.BlockSpec((D, DOUT), lambda i: (0, 0)),
            pl.BlockSpec((D, DOUT), lambda i: (0, 0)),
        ],
        out_specs=[
            pl.BlockSpec((N // 10, DOUT), lambda i: (i, 0)),
            pl.BlockSpec((N // 10, DOUT), lambda i: (i, 0)),
        ],
        out_shape=[
            jax.ShapeDtypeStruct((N, DOUT), jnp.float32),
            jax.ShapeDtypeStruct((N, DOUT), jnp.float32),
        ],
    )(x, wr, ws)


NB = 20
NWAVE = NCH // NB


def _gather_sum(dst_idx, src_idx, xr, xs):
    mesh = plsc.VectorSubcoreMesh(core_axis_name="c", subcore_axis_name="s")

    @functools.partial(
        pl.kernel,
        out_type=jax.ShapeDtypeStruct((E_PAD, DOUT), jnp.float32),
        mesh=mesh,
        scratch_types=[
            pltpu.VMEM((NCH * CH,), jnp.int32),
            pltpu.VMEM((NCH * CH,), jnp.int32),
            pltpu.VMEM((NB * CH, DOUT), jnp.float32),
            pltpu.VMEM_SHARED((N, DOUT), jnp.float32),
            pltpu.VMEM_SHARED((N, DOUT), jnp.float32),
            pltpu.SemaphoreType.DMA,
        ],
        compiler_params=pltpu.CompilerParams(use_tc_tiling_on_sc=False),
    )
    def sc_kernel(dst_hbm, src_hbm, xr_hbm, xs_hbm, g_hbm,
                  idxd, idxs, acc, xr_sh, xs_sh, sem):
        sid = lax.axis_index("s")
        wid = sid * NC + lax.axis_index("c")
        rows = N // NS
        pltpu.sync_copy(
            xr_hbm.at[pl.ds(sid * rows, rows)],
            xr_sh.at[pl.ds(sid * rows, rows)],
        )
        pltpu.sync_copy(
            xs_hbm.at[pl.ds(sid * rows, rows)],
            xs_sh.at[pl.ds(sid * rows, rows)],
        )
        pltpu.sync_copy(dst_hbm.at[wid], idxd)
        pltpu.sync_copy(src_hbm.at[wid], idxs)
        plsc.subcore_barrier()
        base = wid * (NCH * CH)
        wch = NB * CH

        def wave(w, carry):
            off = base + w * wch
            pltpu.async_copy(
                xr_sh.at[idxd.at[pl.ds(w * wch, wch)]], acc, sem
            ).wait()
            pltpu.async_copy(
                xs_sh.at[idxs.at[pl.ds(w * wch, wch)]], acc, sem, add=True
            ).wait()
            pltpu.sync_copy(acc, g_hbm.at[pl.ds(off, wch)])
            return carry

        lax.fori_loop(0, NWAVE, wave, 0)

    return sc_kernel(dst_idx, src_idx, xr, xs)


def _final_body(ea_ref, g_ref, we_ref, wu_ref, u_ref, b_ref, out_ref):
    const = u_ref[0, 0] * wu_ref[...] + b_ref[...]
    packed = (
        jnp.dot(ea_ref[...], we_ref[...], preferred_element_type=jnp.float32)
        + g_ref[...]
        + const
    )
    rows = packed.shape[0]
    rep = jnp.broadcast_to(packed[:, None, :], (rows, 8, 128)).reshape(
        rows * 8, 128
    )
    lane = jax.lax.broadcasted_iota(jnp.int32, (rows * 8, 128), 1)
    row = jax.lax.broadcasted_iota(jnp.int32, (rows * 8, 128), 0)
    mask = (lane // DOUT) == (row % 8)
    sel = (
        jax.lax.broadcasted_iota(jnp.int32, (128, DOUT), 0) % DOUT
        == jax.lax.broadcasted_iota(jnp.int32, (128, DOUT), 1)
    ).astype(jnp.float32)
    narrow = jnp.dot(
        jnp.where(mask, rep, 0.0), sel, preferred_element_type=jnp.float32
    )
    out_ref[...] = narrow


def _finalize(ea2, g2, we_big, wu_big, u, b_big):
    blk = 6400
    return pl.pallas_call(
        _final_body,
        grid=(E // blk,),
        in_specs=[
            pl.BlockSpec((blk // 8, 8 * DE), lambda i: (i, 0)),
            pl.BlockSpec((blk // 8, 8 * DOUT), lambda i: (i, 0)),
            pl.BlockSpec((8 * DE, 8 * DOUT), lambda i: (0, 0)),
            pl.BlockSpec((1, 8 * DOUT), lambda i: (0, 0)),
            pl.BlockSpec((1, 1), lambda i: (0, 0), memory_space=pltpu.SMEM),
            pl.BlockSpec((1, 8 * DOUT), lambda i: (0, 0)),
        ],
        out_specs=pl.BlockSpec((blk, DOUT), lambda i: (i, 0)),
        out_shape=jax.ShapeDtypeStruct((E, DOUT), jnp.float32),
    )(ea2, g2, we_big, wu_big, u, b_big)


def kernel(x, edge_index, edge_attr, u, W, b):
    wr = W[DE:DE + D]
    ws = W[DE + D:DE + 2 * D]
    we = W[:DE]
    wu = W[DE + 2 * D:]

    xr, xs = _project(x, wr, ws)

    idx = jnp.zeros((2, E_PAD), jnp.int32).at[:, :E].set(edge_index)
    dst_idx = idx[1].reshape(NW, NCH * CH)
    src_idx = idx[0].reshape(NW, NCH * CH)

    g_pad = _gather_sum(dst_idx, src_idx, xr, xs)

    eye8 = jnp.eye(8, dtype=jnp.float32)
    out2 = _finalize(
        edge_attr.reshape(E // 8, 8 * DE),
        g_pad.reshape(E_PAD // 8, 8 * DOUT),
        jnp.kron(eye8, we),
        jnp.tile(wu, (1, 8)),
        u.reshape(1, 1),
        jnp.tile(b.reshape(1, DOUT), (1, 8)),
    )
    return out2


# speedup vs baseline: 7.6359x; 1.3207x over previous
import functools

import jax
import jax.numpy as jnp
from jax import lax
from jax.experimental import pallas as pl
from jax.experimental.pallas import tpu as pltpu
from jax.experimental.pallas import tpu_sc as plsc

N = 10000
E = 320000
D = 128
DE = 16
DOUT = 16

NC = 2
NS = 16
NW = NC * NS
CH = 128
NCH = 80
E_PAD = NW * NCH * CH


def _proj_body(x_ref, wr_ref, ws_ref, xr_ref, xs_ref):
    xb = x_ref[...]
    xr_ref[...] = jnp.dot(xb, wr_ref[...], preferred_element_type=jnp.float32)
    xs_ref[...] = jnp.dot(xb, ws_ref[...], preferred_element_type=jnp.float32)


def _project(x, wr, ws):
    return pl.pallas_call(
        _proj_body,
        grid=(10,),
        in_specs=[
            pl.BlockSpec((N // 10, D), lambda i: (i, 0)),
            pl.BlockSpec((D, DOUT), lambda i: (0, 0)),
            pl.BlockSpec((D, DOUT), lambda i: (0, 0)),
        ],
        out_specs=[
            pl.BlockSpec((N // 10, DOUT), lambda i: (i, 0)),
            pl.BlockSpec((N // 10, DOUT), lambda i: (i, 0)),
        ],
        out_shape=[
            jax.ShapeDtypeStruct((N, DOUT), jnp.float32),
            jax.ShapeDtypeStruct((N, DOUT), jnp.float32),
        ],
    )(x, wr, ws)


NB = 20
NWAVE = NCH // NB


def _gather_sum(dst_idx, src_idx, xr, xs):
    mesh = plsc.VectorSubcoreMesh(core_axis_name="c", subcore_axis_name="s")

    @functools.partial(
        pl.kernel,
        out_type=jax.ShapeDtypeStruct((E_PAD, DOUT), jnp.float32),
        mesh=mesh,
        scratch_types=[
            pltpu.VMEM((NCH * CH,), jnp.int32),
            pltpu.VMEM((NCH * CH,), jnp.int32),
            pltpu.VMEM((NB * CH, DOUT), jnp.float32),
            pltpu.VMEM_SHARED((N, DOUT), jnp.float32),
            pltpu.VMEM_SHARED((N, DOUT), jnp.float32),
            pltpu.SemaphoreType.DMA,
        ],
        compiler_params=pltpu.CompilerParams(use_tc_tiling_on_sc=False),
    )
    def sc_kernel(dst_hbm, src_hbm, xr_hbm, xs_hbm, g_hbm,
                  idxd, idxs, acc, xr_sh, xs_sh, sem):
        sid = lax.axis_index("s")
        wid = sid * NC + lax.axis_index("c")
        rows = N // NS
        pltpu.sync_copy(
            xr_hbm.at[pl.ds(sid * rows, rows)],
            xr_sh.at[pl.ds(sid * rows, rows)],
        )
        pltpu.sync_copy(
            xs_hbm.at[pl.ds(sid * rows, rows)],
            xs_sh.at[pl.ds(sid * rows, rows)],
        )
        pltpu.sync_copy(dst_hbm.at[wid], idxd)
        pltpu.sync_copy(src_hbm.at[wid], idxs)
        plsc.subcore_barrier()
        base = wid * (NCH * CH)
        wch = NB * CH

        def wave(w, carry):
            off = base + w * wch
            pltpu.async_copy(
                xr_sh.at[idxd.at[pl.ds(w * wch, wch)]], acc, sem
            ).wait()
            pltpu.async_copy(
                xs_sh.at[idxs.at[pl.ds(w * wch, wch)]], acc, sem, add=True
            ).wait()
            pltpu.sync_copy(acc, g_hbm.at[pl.ds(off, wch)])
            return carry

        lax.fori_loop(0, NWAVE, wave, 0)

    return sc_kernel(dst_idx, src_idx, xr, xs)


def _final_body(eat_ref, gt_ref, wet_ref, wut_ref, u_ref, bt_ref, out_ref):
    const = u_ref[0, 0] * wut_ref[...] + bt_ref[...]
    out_ref[...] = (
        jnp.dot(wet_ref[...], eat_ref[...], preferred_element_type=jnp.float32)
        + gt_ref[...]
        + const
    )


def _finalize(ea_t, g_t, we_t, wu_t, u, b_t):
    blk = 32000
    return pl.pallas_call(
        _final_body,
        grid=(E // blk,),
        in_specs=[
            pl.BlockSpec((DE, blk), lambda i: (0, i)),
            pl.BlockSpec((DOUT, blk), lambda i: (0, i)),
            pl.BlockSpec((DOUT, DE), lambda i: (0, 0)),
            pl.BlockSpec((DOUT, 1), lambda i: (0, 0)),
            pl.BlockSpec((1, 1), lambda i: (0, 0), memory_space=pltpu.SMEM),
            pl.BlockSpec((DOUT, 1), lambda i: (0, 0)),
        ],
        out_specs=pl.BlockSpec((DOUT, blk), lambda i: (0, i)),
        out_shape=jax.ShapeDtypeStruct((DOUT, E), jnp.float32),
    )(ea_t, g_t, we_t, wu_t, u, b_t)


def kernel(x, edge_index, edge_attr, u, W, b):
    wr = W[DE:DE + D]
    ws = W[DE + D:DE + 2 * D]
    we = W[:DE]
    wu = W[DE + 2 * D:]

    xr, xs = _project(x, wr, ws)

    idx = jnp.zeros((2, E_PAD), jnp.int32).at[:, :E].set(edge_index)
    dst_idx = idx[1].reshape(NW, NCH * CH)
    src_idx = idx[0].reshape(NW, NCH * CH)

    g_pad = _gather_sum(dst_idx, src_idx, xr, xs)

    out_t = _finalize(
        edge_attr.T,
        g_pad.T,
        we.T,
        wu.reshape(DOUT, 1),
        u.reshape(1, 1),
        b.reshape(DOUT, 1),
    )
    return out_t.T


# speedup vs baseline: 9.8226x; 1.2864x over previous
import functools

import jax
import jax.numpy as jnp
from jax import lax
from jax.experimental import pallas as pl
from jax.experimental.pallas import tpu as pltpu
from jax.experimental.pallas import tpu_sc as plsc

N = 10000
E = 320000
D = 128
DE = 16
DOUT = 16

NC = 2
NS = 16
NW = NC * NS
CH = 128
NCH = 80
E_PAD = NW * NCH * CH


def _proj_body(x_ref, wr_ref, ws_ref, xr_ref, xs_ref):
    xb = x_ref[...]
    xr_ref[...] = jnp.dot(xb, wr_ref[...], preferred_element_type=jnp.float32)
    xs_ref[...] = jnp.dot(xb, ws_ref[...], preferred_element_type=jnp.float32)


def _project(x, wr, ws):
    return pl.pallas_call(
        _proj_body,
        grid=(10,),
        in_specs=[
            pl.BlockSpec((N // 10, D), lambda i: (i, 0)),
            pl.BlockSpec((D, DOUT), lambda i: (0, 0)),
            pl.BlockSpec((D, DOUT), lambda i: (0, 0)),
        ],
        out_specs=[
            pl.BlockSpec((N // 10, DOUT), lambda i: (i, 0)),
            pl.BlockSpec((N // 10, DOUT), lambda i: (i, 0)),
        ],
        out_shape=[
            jax.ShapeDtypeStruct((N, DOUT), jnp.float32),
            jax.ShapeDtypeStruct((N, DOUT), jnp.float32),
        ],
    )(x, wr, ws)


NB = 20
NWAVE = NCH // NB


def _gather_sum(dst_idx, src_idx, xr, xs):
    mesh = plsc.VectorSubcoreMesh(core_axis_name="c", subcore_axis_name="s")
    wch = NB * CH

    @functools.partial(
        pl.kernel,
        out_type=jax.ShapeDtypeStruct((DOUT, E_PAD), jnp.float32),
        mesh=mesh,
        scratch_types=[
            pltpu.VMEM((NCH * CH,), jnp.int32),
            pltpu.VMEM((NCH * CH,), jnp.int32),
            pltpu.VMEM((wch, DOUT), jnp.float32),
            pltpu.VMEM((DOUT, wch), jnp.float32),
            pltpu.VMEM_SHARED((N, DOUT), jnp.float32),
            pltpu.VMEM_SHARED((N, DOUT), jnp.float32),
            pltpu.SemaphoreType.DMA,
        ],
        compiler_params=pltpu.CompilerParams(
            use_tc_tiling_on_sc=False, needs_layout_passes=False
        ),
    )
    def sc_kernel(dst_hbm, src_hbm, xr_hbm, xs_hbm, gt_hbm,
                  idxd, idxs, acc, acc_t, xr_sh, xs_sh, sem):
        sid = lax.axis_index("s")
        wid = sid * NC + lax.axis_index("c")
        rows = N // NS
        pltpu.sync_copy(
            xr_hbm.at[pl.ds(sid * rows, rows)],
            xr_sh.at[pl.ds(sid * rows, rows)],
        )
        pltpu.sync_copy(
            xs_hbm.at[pl.ds(sid * rows, rows)],
            xs_sh.at[pl.ds(sid * rows, rows)],
        )
        pltpu.sync_copy(dst_hbm.at[wid], idxd)
        pltpu.sync_copy(src_hbm.at[wid], idxs)
        plsc.subcore_barrier()
        base = wid * (NCH * CH)
        iota = lax.iota(jnp.int32, 16)

        def wave(w, carry):
            off = base + w * wch
            pltpu.async_copy(
                xr_sh.at[idxd.at[pl.ds(w * wch, wch)]], acc, sem
            ).wait()
            pltpu.async_copy(
                xs_sh.at[idxs.at[pl.ds(w * wch, wch)]], acc, sem, add=True
            ).wait()

            def tile16(c, carry2):
                row_idx = c * 16 + iota
                for j in range(DOUT):
                    col_idx = jnp.full((16,), j, jnp.int32)
                    v = plsc.load_gather(acc, [row_idx, col_idx])
                    acc_t[j, pl.ds(c * 16, 16)] = v
                return carry2

            lax.fori_loop(0, wch // 16, tile16, 0)
            pltpu.sync_copy(acc_t, gt_hbm.at[:, pl.ds(off, wch)])
            return carry

        lax.fori_loop(0, NWAVE, wave, 0)

    return sc_kernel(dst_idx, src_idx, xr, xs)


def _final_body(eat_ref, gt_ref, wet_ref, wut_ref, u_ref, bt_ref, out_ref):
    const = u_ref[0, 0] * wut_ref[...] + bt_ref[...]
    out_ref[...] = (
        jnp.dot(wet_ref[...], eat_ref[...], preferred_element_type=jnp.float32)
        + gt_ref[...]
        + const
    )


def _finalize(ea_t, g_t, we_t, wu_t, u, b_t):
    blk = 32000
    return pl.pallas_call(
        _final_body,
        grid=(E // blk,),
        in_specs=[
            pl.BlockSpec((DE, blk), lambda i: (0, i)),
            pl.BlockSpec((DOUT, blk), lambda i: (0, i)),
            pl.BlockSpec((DOUT, DE), lambda i: (0, 0)),
            pl.BlockSpec((DOUT, 1), lambda i: (0, 0)),
            pl.BlockSpec((1, 1), lambda i: (0, 0), memory_space=pltpu.SMEM),
            pl.BlockSpec((DOUT, 1), lambda i: (0, 0)),
        ],
        out_specs=pl.BlockSpec((DOUT, blk), lambda i: (0, i)),
        out_shape=jax.ShapeDtypeStruct((DOUT, E), jnp.float32),
    )(ea_t, g_t, we_t, wu_t, u, b_t)


def kernel(x, edge_index, edge_attr, u, W, b):
    wr = W[DE:DE + D]
    ws = W[DE + D:DE + 2 * D]
    we = W[:DE]
    wu = W[DE + 2 * D:]

    xr, xs = _project(x, wr, ws)

    idx = jnp.zeros((2, E_PAD), jnp.int32).at[:, :E].set(edge_index)
    dst_idx = idx[1].reshape(NW, NCH * CH)
    src_idx = idx[0].reshape(NW, NCH * CH)

    g_t = _gather_sum(dst_idx, src_idx, xr, xs)

    out_t = _finalize(
        edge_attr.T,
        g_t,
        we.T,
        wu.reshape(DOUT, 1),
        u.reshape(1, 1),
        b.reshape(DOUT, 1),
    )
    return out_t.T


# speedup vs baseline: 11.9506x; 1.2166x over previous
import functools

import jax
import jax.numpy as jnp
from jax import lax
from jax.experimental import pallas as pl
from jax.experimental.pallas import tpu as pltpu
from jax.experimental.pallas import tpu_sc as plsc

N = 10000
E = 320000
D = 128
DE = 16
DOUT = 16

NC = 2
NS = 16
NW = NC * NS
CH = 128
NCH = 80
E_PAD = NW * NCH * CH


def _proj_body(x_ref, wr_ref, ws_ref, xr_ref, xs_ref):
    xb = x_ref[...]
    xr_ref[...] = jnp.dot(xb, wr_ref[...], preferred_element_type=jnp.float32)
    xs_ref[...] = jnp.dot(xb, ws_ref[...], preferred_element_type=jnp.float32)


def _project(x, wr, ws):
    return pl.pallas_call(
        _proj_body,
        grid=(10,),
        in_specs=[
            pl.BlockSpec((N // 10, D), lambda i: (i, 0)),
            pl.BlockSpec((D, DOUT), lambda i: (0, 0)),
            pl.BlockSpec((D, DOUT), lambda i: (0, 0)),
        ],
        out_specs=[
            pl.BlockSpec((N // 10, DOUT), lambda i: (i, 0)),
            pl.BlockSpec((N // 10, DOUT), lambda i: (i, 0)),
        ],
        out_shape=[
            jax.ShapeDtypeStruct((N, DOUT), jnp.float32),
            jax.ShapeDtypeStruct((N, DOUT), jnp.float32),
        ],
    )(x, wr, ws)


NB = 20
NWAVE = NCH // NB


def _gather_sum(dst_idx, src_idx, xr, xs):
    mesh = plsc.VectorSubcoreMesh(core_axis_name="c", subcore_axis_name="s")
    wch = NB * CH

    @functools.partial(
        pl.kernel,
        out_type=jax.ShapeDtypeStruct((DOUT, E_PAD), jnp.float32),
        mesh=mesh,
        scratch_types=[
            pltpu.VMEM((NCH * CH,), jnp.int32),
            pltpu.VMEM((NCH * CH,), jnp.int32),
            pltpu.VMEM((wch, DOUT), jnp.float32),
            pltpu.VMEM((DOUT, wch), jnp.float32),
            pltpu.VMEM_SHARED((N, DOUT), jnp.float32),
            pltpu.VMEM_SHARED((N, DOUT), jnp.float32),
            pltpu.SemaphoreType.DMA,
        ],
        compiler_params=pltpu.CompilerParams(
            use_tc_tiling_on_sc=False, needs_layout_passes=False
        ),
    )
    def sc_kernel(dst_hbm, src_hbm, xr_hbm, xs_hbm, gt_hbm,
                  idxd, idxs, acc, acc_t, xr_sh, xs_sh, sem):
        sid = lax.axis_index("s")
        wid = sid * NC + lax.axis_index("c")
        rows = N // NS
        pltpu.sync_copy(
            xr_hbm.at[pl.ds(sid * rows, rows)],
            xr_sh.at[pl.ds(sid * rows, rows)],
        )
        pltpu.sync_copy(
            xs_hbm.at[pl.ds(sid * rows, rows)],
            xs_sh.at[pl.ds(sid * rows, rows)],
        )
        pltpu.sync_copy(dst_hbm.at[wid], idxd)
        pltpu.sync_copy(src_hbm.at[wid], idxs)
        plsc.subcore_barrier()
        base = wid * (NCH * CH)
        iota = lax.iota(jnp.int32, 16)
        diag = [lax.rem(iota + d, 16) for d in range(DOUT)]

        def wave(w, carry):
            off = base + w * wch
            pltpu.async_copy(
                xr_sh.at[idxd.at[pl.ds(w * wch, wch)]], acc, sem
            ).wait()
            pltpu.async_copy(
                xs_sh.at[idxs.at[pl.ds(w * wch, wch)]], acc, sem, add=True
            ).wait()

            def tile16(c, carry2):
                row_idx = c * 16 + iota
                for d in range(DOUT):
                    v = plsc.load_gather(acc, [row_idx, diag[d]])
                    plsc.store_scatter(acc_t, [diag[d], row_idx], v)
                return carry2

            lax.fori_loop(0, wch // 16, tile16, 0)
            pltpu.sync_copy(acc_t, gt_hbm.at[:, pl.ds(off, wch)])
            return carry

        lax.fori_loop(0, NWAVE, wave, 0)

    return sc_kernel(dst_idx, src_idx, xr, xs)


def _final_body(eat_ref, gt_ref, wet_ref, wut_ref, u_ref, bt_ref, out_ref):
    const = u_ref[0, 0] * wut_ref[...] + bt_ref[...]
    out_ref[...] = (
        jnp.dot(wet_ref[...], eat_ref[...], preferred_element_type=jnp.float32)
        + gt_ref[...]
        + const
    )


def _finalize(ea_t, g_t, we_t, wu_t, u, b_t):
    blk = 32000
    return pl.pallas_call(
        _final_body,
        grid=(E // blk,),
        in_specs=[
            pl.BlockSpec((DE, blk), lambda i: (0, i)),
            pl.BlockSpec((DOUT, blk), lambda i: (0, i)),
            pl.BlockSpec((DOUT, DE), lambda i: (0, 0)),
            pl.BlockSpec((DOUT, 1), lambda i: (0, 0)),
            pl.BlockSpec((1, 1), lambda i: (0, 0), memory_space=pltpu.SMEM),
            pl.BlockSpec((DOUT, 1), lambda i: (0, 0)),
        ],
        out_specs=pl.BlockSpec((DOUT, blk), lambda i: (0, i)),
        out_shape=jax.ShapeDtypeStruct((DOUT, E), jnp.float32),
    )(ea_t, g_t, we_t, wu_t, u, b_t)


def kernel(x, edge_index, edge_attr, u, W, b):
    wr = W[DE:DE + D]
    ws = W[DE + D:DE + 2 * D]
    we = W[:DE]
    wu = W[DE + 2 * D:]

    xr, xs = _project(x, wr, ws)

    idx = jnp.zeros((2, E_PAD), jnp.int32).at[:, :E].set(edge_index)
    dst_idx = idx[1].reshape(NW, NCH * CH)
    src_idx = idx[0].reshape(NW, NCH * CH)

    g_t = _gather_sum(dst_idx, src_idx, xr, xs)

    out_t = _finalize(
        edge_attr.T,
        g_t,
        we.T,
        wu.reshape(DOUT, 1),
        u.reshape(1, 1),
        b.reshape(DOUT, 1),
    )
    return out_t.T
